# baseline (speedup 1.0000x reference)
import functools

import jax
import jax.numpy as jnp
from jax import lax
from jax.experimental import pallas as pl

_N = 320000
_D = 128
_S = 10000
_BETA = 0.8
_W = 128
_B = 6400
_NB = _N // _B
_SEG_PAD = ((_S + _W - 1) // _W) * _W


def _body(ids_ref, ts_ref, msg_ref, agg_ref, smax_ref, cnt_ref, wsum_ref):
    p = pl.program_id(0)
    j = pl.program_id(1)

    id_row = ids_ref[0]
    t_row = ts_ref[0]

    w0 = jnp.min(id_row) // _W
    w1 = jnp.max(id_row) // _W

    seg_iota = lax.broadcasted_iota(jnp.int32, (_W, 1), 0)

    @pl.when((p == 0) & (j == 0))
    def _init0():
        smax_ref[...] = jnp.zeros_like(smax_ref)
        cnt_ref[...] = jnp.zeros_like(cnt_ref)

    @pl.when((p == 1) & (j == 0))
    def _init1():
        wsum_ref[...] = jnp.zeros_like(wsum_ref)
        agg_ref[...] = jnp.zeros_like(agg_ref)

    @pl.when(p == 0)
    def _pass_max():
        def body(w, carry):
            base = w * _W
            eq = (id_row - base) == seg_iota
            onehot = jnp.where(eq, 1.0, 0.0)
            tmask = jnp.where(eq, jnp.broadcast_to(t_row, eq.shape), 0.0)
            mx = jnp.max(tmask, axis=1, keepdims=True)
            ct = jnp.sum(onehot, axis=1, keepdims=True)
            smax_ref[pl.ds(base, _W), :] = jnp.maximum(
                smax_ref[pl.ds(base, _W), :], mx)
            cnt_ref[pl.ds(base, _W), :] += ct
            return carry

        lax.fori_loop(w0, w1 + 1, body, 0)

    @pl.when(p == 1)
    def _pass_acc():
        msg = msg_ref[...]

        def body(w, carry):
            base = w * _W
            eq = (id_row - base) == seg_iota
            onehot = jnp.where(eq, 1.0, 0.0)
            smax_win = smax_ref[pl.ds(base, _W), :]
            last_t = jnp.sum(onehot * smax_win, axis=0, keepdims=True)
            arg = jnp.minimum(_BETA * (t_row - last_t), 0.0)
            w_row = jnp.exp(arg)
            wo = onehot * w_row
            wsum_ref[pl.ds(base, _W), :] += jnp.sum(wo, axis=1, keepdims=True)
            agg_ref[pl.ds(base, _W), :] += jnp.dot(
                wo, msg, preferred_element_type=jnp.float32)
            return carry

        lax.fori_loop(w0, w1 + 1, body, 0)

    @pl.when((p == 1) & (j == _NB - 1))
    def _finish():
        cnt = cnt_ref[...]
        wsum = wsum_ref[...]
        inv = jnp.where(cnt > 0.0, 1.0 / wsum, 0.0)
        agg_ref[...] = agg_ref[...] * inv


@functools.partial(jax.jit, static_argnames=("interpret",))
def _run(node_ids, messages, timestamps, interpret=False):
    ids3 = node_ids.reshape(_NB, 1, _B).astype(jnp.int32)
    ts3 = timestamps.reshape(_NB, 1, _B)

    grid = (2, _NB)
    out = pl.pallas_call(
        _body,
        grid=grid,
        in_specs=[
            pl.BlockSpec((1, 1, _B), lambda p, j: (j, 0, 0)),
            pl.BlockSpec((1, 1, _B), lambda p, j: (j, 0, 0)),
            pl.BlockSpec((_B, _D), lambda p, j: (p * j, 0)),
        ],
        out_specs=[
            pl.BlockSpec((_SEG_PAD, _D), lambda p, j: (0, 0)),
            pl.BlockSpec((_SEG_PAD, 1), lambda p, j: (0, 0)),
            pl.BlockSpec((_SEG_PAD, 1), lambda p, j: (0, 0)),
            pl.BlockSpec((_SEG_PAD, 1), lambda p, j: (0, 0)),
        ],
        out_shape=[
            jax.ShapeDtypeStruct((_SEG_PAD, _D), jnp.float32),
            jax.ShapeDtypeStruct((_SEG_PAD, 1), jnp.float32),
            jax.ShapeDtypeStruct((_SEG_PAD, 1), jnp.float32),
            jax.ShapeDtypeStruct((_SEG_PAD, 1), jnp.float32),
        ],
        interpret=interpret,
    )(ids3, ts3, messages)

    agg, smax, cnt, _ = out
    agg = agg[:_S]
    seg_max = smax[:_S, 0]
    present = cnt[:_S, 0] > 0.0
    return agg, seg_max, present


def kernel(node_ids, messages, timestamps):
    return _run(node_ids, messages, timestamps)


# speedup vs baseline: 1.3982x; 1.3982x over previous
import functools

import jax
import jax.numpy as jnp
from jax import lax
from jax.experimental import pallas as pl

_N = 320000
_D = 128
_S = 10000
_BETA = 0.8
_W = 128
_B = 3200
_NB = _N // _B
_SEG_PAD = ((_S + _W - 1) // _W) * _W


def _body(ids_ref, ts_ref, msg_ref, agg_ref, smax_ref, cnt_ref, wsum_ref):
    p = pl.program_id(0)
    j = pl.program_id(1)

    id_row = ids_ref[0]
    t_row = ts_ref[0]

    base0 = jnp.minimum((jnp.min(id_row) // 8) * 8, _SEG_PAD - _W)
    nw = (jnp.max(id_row) - base0) // _W + 1

    seg_iota = lax.broadcasted_iota(jnp.int32, (_W, 1), 0)

    @pl.when((p == 0) & (j == 0))
    def _init0():
        smax_ref[...] = jnp.zeros_like(smax_ref)
        cnt_ref[...] = jnp.zeros_like(cnt_ref)

    @pl.when((p == 1) & (j == 0))
    def _init1():
        wsum_ref[...] = jnp.zeros_like(wsum_ref)
        agg_ref[...] = jnp.zeros_like(agg_ref)

    def _window(w):
        start = base0 + w * _W
        base = jnp.minimum(start, _SEG_PAD - _W)
        eq = ((id_row - base) == seg_iota) & (id_row >= start)
        return base, eq

    @pl.when(p == 0)
    def _pass_max():
        def body(w, carry):
            base, eq = _window(w)
            onehot = jnp.where(eq, 1.0, 0.0)
            tmask = jnp.where(eq, jnp.broadcast_to(t_row, eq.shape), 0.0)
            mx = jnp.max(tmask, axis=1, keepdims=True)
            ct = jnp.sum(onehot, axis=1, keepdims=True)
            smax_ref[pl.ds(base, _W), :] = jnp.maximum(
                smax_ref[pl.ds(base, _W), :], mx)
            cnt_ref[pl.ds(base, _W), :] += ct
            return carry

        lax.fori_loop(0, nw, body, 0)

    @pl.when(p == 1)
    def _pass_acc():
        msg = msg_ref[...]

        def body(w, carry):
            base, eq = _window(w)
            onehot = jnp.where(eq, 1.0, 0.0)
            smax_win = smax_ref[pl.ds(base, _W), :]
            last_t = jnp.sum(onehot * smax_win, axis=0, keepdims=True)
            arg = jnp.minimum(_BETA * (t_row - last_t), 0.0)
            w_row = jnp.exp(arg)
            wo = onehot * w_row
            wsum_ref[pl.ds(base, _W), :] += jnp.sum(wo, axis=1, keepdims=True)
            agg_ref[pl.ds(base, _W), :] += jnp.dot(
                wo, msg, preferred_element_type=jnp.float32)
            return carry

        lax.fori_loop(0, nw, body, 0)

    @pl.when((p == 1) & (j == _NB - 1))
    def _finish():
        cnt = cnt_ref[...]
        wsum = wsum_ref[...]
        inv = jnp.where(cnt > 0.0, 1.0 / wsum, 0.0)
        agg_ref[...] = agg_ref[...] * inv


@functools.partial(jax.jit, static_argnames=("interpret",))
def _run(node_ids, messages, timestamps, interpret=False):
    ids3 = node_ids.reshape(_NB, 1, _B).astype(jnp.int32)
    ts3 = timestamps.reshape(_NB, 1, _B)

    grid = (2, _NB)
    out = pl.pallas_call(
        _body,
        grid=grid,
        in_specs=[
            pl.BlockSpec((1, 1, _B), lambda p, j: (j, 0, 0)),
            pl.BlockSpec((1, 1, _B), lambda p, j: (j, 0, 0)),
            pl.BlockSpec((_B, _D), lambda p, j: (p * j, 0)),
        ],
        out_specs=[
            pl.BlockSpec((_SEG_PAD, _D), lambda p, j: (0, 0)),
            pl.BlockSpec((_SEG_PAD, 1), lambda p, j: (0, 0)),
            pl.BlockSpec((_SEG_PAD, 1), lambda p, j: (0, 0)),
            pl.BlockSpec((_SEG_PAD, 1), lambda p, j: (0, 0)),
        ],
        out_shape=[
            jax.ShapeDtypeStruct((_SEG_PAD, _D), jnp.float32),
            jax.ShapeDtypeStruct((_SEG_PAD, 1), jnp.float32),
            jax.ShapeDtypeStruct((_SEG_PAD, 1), jnp.float32),
            jax.ShapeDtypeStruct((_SEG_PAD, 1), jnp.float32),
        ],
        interpret=interpret,
    )(ids3, ts3, messages)

    agg, smax, cnt, _ = out
    agg = agg[:_S]
    seg_max = smax[:_S, 0]
    present = cnt[:_S, 0] > 0.0
    return agg, seg_max, present


def kernel(node_ids, messages, timestamps):
    return _run(node_ids, messages, timestamps)


# speedup vs baseline: 1.5389x; 1.1006x over previous
import functools

import jax
import jax.numpy as jnp
from jax import lax
from jax.experimental import pallas as pl

_N = 320000
_D = 128
_S = 10000
_BETA = 0.8
_W = 128
_B = 3200
_NB = _N // _B
_SEG_PAD = ((_S + _W - 1) // _W) * _W


def _body(ids_ref, ts_ref, msg_ref, agg_ref, smax_ref, wsum_ref):
    p = pl.program_id(0)
    j = pl.program_id(1)

    id_row = ids_ref[0]
    t_row = ts_ref[0]

    base0 = jnp.minimum((jnp.min(id_row) // 8) * 8, _SEG_PAD - _W)
    nw = (jnp.max(id_row) - base0) // _W + 1

    seg_iota = lax.broadcasted_iota(jnp.int32, (_W, 1), 0)

    @pl.when((p == 0) & (j == 0))
    def _init0():
        smax_ref[...] = jnp.zeros_like(smax_ref)

    @pl.when((p == 1) & (j == 0))
    def _init1():
        wsum_ref[...] = jnp.zeros_like(wsum_ref)
        agg_ref[...] = jnp.zeros_like(agg_ref)

    def _window(w):
        start = base0 + w * _W
        base = jnp.minimum(start, _SEG_PAD - _W)
        rel = jnp.where(id_row >= start, id_row - base, -1)
        eq = rel == seg_iota
        return base, eq

    @pl.when(p == 0)
    def _pass_max():
        def body(w, carry):
            base, eq = _window(w)
            tmask = jnp.where(eq, jnp.broadcast_to(t_row, eq.shape), 0.0)
            mx = jnp.max(tmask, axis=1, keepdims=True)
            smax_ref[pl.ds(base, _W), :] = jnp.maximum(
                smax_ref[pl.ds(base, _W), :], mx)
            return carry

        lax.fori_loop(0, nw, body, 0)

    @pl.when(p == 1)
    def _pass_acc():
        msg = msg_ref[...]
        ones_col = jnp.ones((_B, 1), jnp.float32)

        def body(w, carry):
            base, eq = _window(w)
            onehot = jnp.where(eq, 1.0, 0.0)
            smax_win = smax_ref[pl.ds(base, _W), :]
            last_t = lax.dot_general(
                smax_win, onehot, (((0,), (0,)), ((), ())),
                preferred_element_type=jnp.float32)
            arg = jnp.minimum(_BETA * (t_row - last_t), 0.0)
            w_row = jnp.exp(arg)
            wo = onehot * w_row
            wsum_ref[pl.ds(base, _W), :] += jnp.dot(
                wo, ones_col, preferred_element_type=jnp.float32)
            agg_ref[pl.ds(base, _W), :] += jnp.dot(
                wo, msg, preferred_element_type=jnp.float32)
            return carry

        lax.fori_loop(0, nw, body, 0)

    @pl.when((p == 1) & (j == _NB - 1))
    def _finish():
        wsum = wsum_ref[...]
        inv = jnp.where(wsum > 0.0, 1.0 / wsum, 0.0)
        agg_ref[...] = agg_ref[...] * inv


@functools.partial(jax.jit, static_argnames=("interpret",))
def _run(node_ids, messages, timestamps, interpret=False):
    ids3 = node_ids.reshape(_NB, 1, _B).astype(jnp.int32)
    ts3 = timestamps.reshape(_NB, 1, _B)

    grid = (2, _NB)
    out = pl.pallas_call(
        _body,
        grid=grid,
        in_specs=[
            pl.BlockSpec((1, 1, _B), lambda p, j: (j, 0, 0)),
            pl.BlockSpec((1, 1, _B), lambda p, j: (j, 0, 0)),
            pl.BlockSpec((_B, _D), lambda p, j: (p * j, 0)),
        ],
        out_specs=[
            pl.BlockSpec((_SEG_PAD, _D), lambda p, j: (0, 0)),
            pl.BlockSpec((_SEG_PAD, 1), lambda p, j: (0, 0)),
            pl.BlockSpec((_SEG_PAD, 1), lambda p, j: (0, 0)),
        ],
        out_shape=[
            jax.ShapeDtypeStruct((_SEG_PAD, _D), jnp.float32),
            jax.ShapeDtypeStruct((_SEG_PAD, 1), jnp.float32),
            jax.ShapeDtypeStruct((_SEG_PAD, 1), jnp.float32),
        ],
        interpret=interpret,
    )(ids3, ts3, messages)

    agg, smax, wsum = out
    agg = agg[:_S]
    seg_max = smax[:_S, 0]
    present = wsum[:_S, 0] > 0.0
    return agg, seg_max, present


def kernel(node_ids, messages, timestamps):
    return _run(node_ids, messages, timestamps)
